# Initial kernel scaffold; baseline (speedup 1.0000x reference)
import jax
import jax.numpy as jnp
from jax.experimental import pallas as pl


def kernel(x, edge_index, batch, W1_0, b1_0, W2_0, b2_0, gamma_0, beta_0, W1_1, b1_1, W2_1, b2_1, gamma_1, beta_1, W1_2, b1_2, W2_2, b2_2, gamma_2, beta_2):
    raise NotImplementedError("write your pallas kernel here")


# speedup vs baseline: 2.7921x; 2.7921x over previous
import functools

import jax
import jax.numpy as jnp
from jax import lax
from jax.experimental import pallas as pl
from jax.experimental.pallas import tpu as pltpu
from jax.experimental.pallas import tpu_sc as plsc

N = 10000
E = 160000
D = 256
DH = 128
L = 3
G = 64
BN_EPS = 1e-5

NSUB = 16
BLK = 128
BLOCKS_PER_TILE = 80
EPT = BLOCKS_PER_TILE * BLK
EPAD = NSUB * EPT
NPAD = 10112
ZROWS = NPAD // NSUB

R = 2000


@functools.cache
def _make_agg_sc():
  mesh = plsc.VectorSubcoreMesh(core_axis_name="c", subcore_axis_name="s")

  @functools.partial(
      pl.kernel,
      mesh=mesh,
      out_type=[
          jax.ShapeDtypeStruct((NPAD, DH), jnp.float32),
          jax.ShapeDtypeStruct((NPAD, DH), jnp.float32),
      ],
      scratch_types=[
          pltpu.VMEM((BLOCKS_PER_TILE, BLK), jnp.int32),
          pltpu.VMEM((BLOCKS_PER_TILE, BLK), jnp.int32),
          pltpu.VMEM((BLK, DH), jnp.float32),
          pltpu.VMEM_SHARED((NPAD, DH), jnp.float32),
          pltpu.SemaphoreType.DMA,
      ],
  )
  def _agg_sc(h0_hbm, h1_hbm, src2_hbm, dst2_hbm, zeros_hbm, out0_hbm, out1_hbm,
              src_v, dst_v, rows_v, acc_sh, sem):
    c = lax.axis_index("c")
    s = lax.axis_index("s")

    pltpu.sync_copy(zeros_hbm.at[pl.ds(s * ZROWS, ZROWS)],
                    acc_sh.at[pl.ds(s * ZROWS, ZROWS)])
    pltpu.sync_copy(src2_hbm.at[pl.ds(s * BLOCKS_PER_TILE, BLOCKS_PER_TILE)], src_v)
    pltpu.sync_copy(dst2_hbm.at[pl.ds(s * BLOCKS_PER_TILE, BLOCKS_PER_TILE)], dst_v)
    plsc.subcore_barrier()

    def body(h_hbm):
        def step(j, carry):
            pltpu.async_copy(h_hbm.at[src_v.at[j]], rows_v, sem).wait()
            pltpu.sync_copy(rows_v, acc_sh.at[dst_v.at[j]], add=True)
            return carry
        lax.fori_loop(0, BLOCKS_PER_TILE, step, 0)

    @pl.when(c == 0)
    def _():
        body(h0_hbm)

    @pl.when(c == 1)
    def _():
        body(h1_hbm)

    plsc.subcore_barrier()

    @pl.when(c == 0)
    def _():
        pltpu.sync_copy(acc_sh.at[pl.ds(s * ZROWS, ZROWS)],
                        out0_hbm.at[pl.ds(s * ZROWS, ZROWS)])

    @pl.when(c == 1)
    def _():
        pltpu.sync_copy(acc_sh.at[pl.ds(s * ZROWS, ZROWS)],
                        out1_hbm.at[pl.ds(s * ZROWS, ZROWS)])

  return _agg_sc


def _agg_call(h0, h1, src2, dst2, zeros):
    return _make_agg_sc()(h0, h1, src2, dst2, zeros)


def _mlp_body(h0, h1, a0, a1, w1, b1, w2, b2, out, ssum, ssq):
    x = jnp.concatenate([h0[...] + a0[...], h1[...] + a1[...]], axis=1)
    t = jnp.dot(x, w1[...], preferred_element_type=jnp.float32) + b1[...]
    t = jnp.maximum(t, 0.0)
    m = jnp.dot(t, w2[...], preferred_element_type=jnp.float32) + b2[...]
    m = jnp.maximum(m, 0.0)
    out[...] = m

    @pl.when(pl.program_id(0) == 0)
    def _():
        ssum[...] = jnp.zeros_like(ssum)
        ssq[...] = jnp.zeros_like(ssq)

    ssum[...] += jnp.sum(m, axis=0, keepdims=True)
    ssq[...] += jnp.sum(m * m, axis=0, keepdims=True)


def _mlp_call(h0, h1, a0, a1, w1, b1, w2, b2):
    row = pl.BlockSpec((R, DH), lambda i: (i, 0))
    full = pl.BlockSpec((D, D), lambda i: (0, 0))
    vec = pl.BlockSpec((1, D), lambda i: (0, 0))
    return pl.pallas_call(
        _mlp_body,
        grid=(N // R,),
        in_specs=[row, row, row, row, full, vec, full, vec],
        out_specs=[pl.BlockSpec((R, D), lambda i: (i, 0)), vec, vec],
        out_shape=[
            jax.ShapeDtypeStruct((N, D), jnp.float32),
            jax.ShapeDtypeStruct((1, D), jnp.float32),
            jax.ShapeDtypeStruct((1, D), jnp.float32),
        ],
    )(h0, h1, a0, a1, w1, b1, w2, b2)


def _bn_body(mraw, ssum, ssq, gamma, beta, batch, m0, m1, pool):
    mean = ssum[...] * (1.0 / N)
    var = ssq[...] * (1.0 / N) - mean * mean
    scale = gamma[...] * lax.rsqrt(var + BN_EPS)
    shift = beta[...] - mean * scale
    m = mraw[...] * scale + shift
    m0[...] = m[:, :DH]
    m1[...] = m[:, DH:]
    oh = (batch[...] == lax.broadcasted_iota(jnp.int32, (R, G), 1))
    oh = oh.astype(jnp.float32)

    @pl.when(pl.program_id(0) == 0)
    def _():
        pool[...] = jnp.zeros_like(pool)

    pool[...] += lax.dot_general(oh, m, (((0,), (0,)), ((), ())),
                                 preferred_element_type=jnp.float32)


def _bn_call(mraw, ssum, ssq, gamma, beta, batch2):
    vec = pl.BlockSpec((1, D), lambda i: (0, 0))
    return pl.pallas_call(
        _bn_body,
        grid=(N // R,),
        in_specs=[
            pl.BlockSpec((R, D), lambda i: (i, 0)),
            vec, vec, vec, vec,
            pl.BlockSpec((R, 1), lambda i: (i, 0)),
        ],
        out_specs=[
            pl.BlockSpec((R, DH), lambda i: (i, 0)),
            pl.BlockSpec((R, DH), lambda i: (i, 0)),
            pl.BlockSpec((G, D), lambda i: (0, 0)),
        ],
        out_shape=[
            jax.ShapeDtypeStruct((N, DH), jnp.float32),
            jax.ShapeDtypeStruct((N, DH), jnp.float32),
            jax.ShapeDtypeStruct((G, D), jnp.float32),
        ],
    )(mraw, ssum, ssq, gamma, beta, batch2)


def kernel(x, edge_index, batch,
           W1_0, b1_0, W2_0, b2_0, gamma_0, beta_0,
           W1_1, b1_1, W2_1, b2_1, gamma_1, beta_1,
           W1_2, b1_2, W2_2, b2_2, gamma_2, beta_2):
    src = edge_index[0].astype(jnp.int32)
    dst = edge_index[1].astype(jnp.int32)
    pad = EPAD - E
    src2 = jnp.concatenate([src, jnp.zeros((pad,), jnp.int32)]).reshape(-1, BLK)
    dst2 = jnp.concatenate([dst, jnp.full((pad,), N, jnp.int32)]).reshape(-1, BLK)
    zeros = jnp.zeros((NPAD, DH), jnp.float32)
    batch2 = batch.reshape(N, 1).astype(jnp.int32)

    params = [
        (W1_0, b1_0.reshape(1, D), W2_0, b2_0.reshape(1, D),
         gamma_0.reshape(1, D), beta_0.reshape(1, D)),
        (W1_1, b1_1.reshape(1, D), W2_1, b2_1.reshape(1, D),
         gamma_1.reshape(1, D), beta_1.reshape(1, D)),
        (W1_2, b1_2.reshape(1, D), W2_2, b2_2.reshape(1, D),
         gamma_2.reshape(1, D), beta_2.reshape(1, D)),
    ]

    h0 = x[:, :DH]
    h1 = x[:, DH:]
    halves = []
    pools = []
    for i in range(L):
        w1, b1, w2, b2, gmm, bta = params[i]
        a0, a1 = _agg_call(h0, h1, src2, dst2, zeros)
        mraw, ssum, ssq = _mlp_call(h0, h1, a0, a1, w1, b1, w2, b2)
        m0, m1, pool = _bn_call(mraw, ssum, ssq, gmm, bta, batch2)
        h0, h1 = m0, m1
        halves.extend([m0, m1])
        pools.append(pool)

    return jnp.concatenate(pools, axis=1), jnp.concatenate(halves, axis=1)
